# baseline (speedup 1.0000x reference)
import jax
import jax.numpy as jnp
from jax.experimental import pallas as pl

N = 10000
E = 160000
H = 16
NSTEP_MP = 3
NSTEP_S2S = 6
NLAYER_S2S = 3


def _proj_body(x_ref, wp_ref, bp_ref, o_ref):
    o_ref[...] = jax.nn.relu(
        jnp.dot(x_ref[...], wp_ref[...], preferred_element_type=jnp.float32)
        + bp_ref[...]
    )


def _gru_cell(xi, h, Wih, Whh, bih, bhh):
    gi = xi @ Wih.T + bih
    gh = h @ Whh.T + bhh
    i_r, i_z, i_n = jnp.split(gi, 3, axis=-1)
    h_r, h_z, h_n = jnp.split(gh, 3, axis=-1)
    r = jax.nn.sigmoid(i_r + h_r)
    z = jax.nn.sigmoid(i_z + h_z)
    n = jnp.tanh(i_n + r * h_n)
    return (1.0 - z) * n + z * h


def _lstm_cell(xi, h, c, Wih, Whh, bih, bhh):
    g = xi @ Wih.T + bih + h @ Whh.T + bhh
    i, f, gg, o = jnp.split(g, 4, axis=-1)
    i = jax.nn.sigmoid(i)
    f = jax.nn.sigmoid(f)
    gg = jnp.tanh(gg)
    o = jax.nn.sigmoid(o)
    c_new = f * c + i * gg
    h_new = o * jnp.tanh(c_new)
    return h_new, c_new


def kernel(x, edge_index, edge_attr, Wp, bp, We1, be1, We2, be2, b_nn, gru_Wih, gru_Whh, gru_bih, gru_bhh, lstm_Wih0, lstm_Wih12, lstm_Whh, lstm_bih, lstm_bhh, Wp1, bp1, Wp2, bp2):
    src = edge_index[0]
    dst = edge_index[1]
    h_nodes = pl.pallas_call(
        _proj_body,
        out_shape=jax.ShapeDtypeStruct((N, H), jnp.float32),
    )(x, Wp, bp.reshape(1, H))
    hidden = h_nodes
    We = jax.nn.relu(edge_attr @ We1 + be1) @ We2 + be2
    We = We.reshape(E, H, H)
    for _ in range(NSTEP_MP):
        msg = jnp.einsum('ei,eij->ej', h_nodes[src], We)
        agg = jax.ops.segment_sum(msg, dst, num_segments=N)
        h_new = jax.nn.relu(agg + b_nn)
        hidden = _gru_cell(h_new, hidden, gru_Wih, gru_Whh, gru_bih, gru_bhh)
        h_nodes = hidden
    q_star = jnp.zeros((1, 2 * H), dtype=x.dtype)
    hs = [jnp.zeros((1, H), dtype=x.dtype) for _ in range(NLAYER_S2S)]
    cs = [jnp.zeros((1, H), dtype=x.dtype) for _ in range(NLAYER_S2S)]
    for _ in range(NSTEP_S2S):
        inp_l = q_star
        for l in range(NLAYER_S2S):
            Wih = lstm_Wih0 if l == 0 else lstm_Wih12[l - 1]
            h_l, c_l = _lstm_cell(inp_l, hs[l], cs[l], Wih, lstm_Whh[l], lstm_bih[l], lstm_bhh[l])
            hs[l] = h_l
            cs[l] = c_l
            inp_l = h_l
        q = inp_l
        e = jnp.sum(h_nodes * q, axis=-1)
        alpha = jax.nn.softmax(e)
        readout = jnp.sum(alpha[:, None] * h_nodes, axis=0, keepdims=True)
        q_star = jnp.concatenate([q, readout], axis=1)
    hid = jax.nn.relu(q_star @ Wp1 + bp1)
    out = hid @ Wp2 + bp2
    return out


# speedup vs baseline: 1.0919x; 1.0919x over previous
import functools

import jax
import jax.numpy as jnp
from jax import lax
from jax.experimental import pallas as pl
from jax.experimental.pallas import tpu as pltpu
from jax.experimental.pallas import tpu_sc as plsc

N = 10000
E = 160000
D_IN = 128
D_E = 16
H = 16
EH = 128
NSTEP_MP = 3
NSTEP_S2S = 6
NLAYER_S2S = 3

_NC = 2
_NS = 16
_NW = _NC * _NS

_CHUNK = 128
_KCH = 40
_BPW = _CHUNK * _KCH
_EPAD = _BPW * _NW
_ET = 2048
_NT_E = _EPAD // _ET
_RPT = N // _NS

_sc_mesh = plsc.VectorSubcoreMesh(core_axis_name="c", subcore_axis_name="s")
_sc_params = pltpu.CompilerParams(use_tc_tiling_on_sc=False)


@functools.partial(
    pl.kernel,
    out_type=jax.ShapeDtypeStruct((_EPAD, H), jnp.float32),
    mesh=_sc_mesh,
    compiler_params=_sc_params,
    scratch_types=[
        pltpu.VMEM((_KCH, _CHUNK), jnp.int32),
        pltpu.VMEM((_BPW, H), jnp.float32),
        pltpu.SemaphoreType.DMA,
    ],
)
def _sc_gather(table_hbm, idx_hbm, out_hbm, idx_v, rows_v, sem):
    wid = lax.axis_index("s") * _NC + lax.axis_index("c")
    pltpu.sync_copy(idx_hbm.at[wid], idx_v)

    def grp(g, carry):
        for b in range(8):
            j = g * 8 + b
            pltpu.async_copy(
                table_hbm.at[idx_v.at[j]], rows_v.at[pl.ds(j * _CHUNK, _CHUNK)], sem
            )
        for b in range(8):
            j = g * 8 + b
            pltpu.make_async_copy(
                table_hbm.at[idx_v.at[j]], rows_v.at[pl.ds(j * _CHUNK, _CHUNK)], sem
            ).wait()
        return carry

    lax.fori_loop(0, _KCH // 8, grp, 0)
    pltpu.sync_copy(rows_v, out_hbm.at[pl.ds(wid * _BPW, _BPW)])


@functools.partial(
    pl.kernel,
    out_type=jax.ShapeDtypeStruct((_NC, N, H), jnp.float32),
    mesh=_sc_mesh,
    compiler_params=_sc_params,
    scratch_types=[
        pltpu.VMEM((_KCH, _CHUNK), jnp.int32),
        pltpu.VMEM((_BPW, H), jnp.float32),
        pltpu.VMEM_SHARED((N, H), jnp.float32),
    ],
)
def _sc_scatter_add(msg_hbm, dst_hbm, zeros_hbm, out_hbm, idx_v, msg_v, acc_sh):
    c = lax.axis_index("c")
    s = lax.axis_index("s")
    wid = s * _NC + c
    pltpu.sync_copy(zeros_hbm.at[pl.ds(s * _RPT, _RPT)], acc_sh.at[pl.ds(s * _RPT, _RPT)])
    pltpu.sync_copy(dst_hbm.at[wid], idx_v)
    pltpu.sync_copy(msg_hbm.at[pl.ds(wid * _BPW, _BPW)], msg_v)
    plsc.subcore_barrier()

    def body(j, carry):
        pltpu.sync_copy(
            msg_v.at[pl.ds(j * _CHUNK, _CHUNK)], acc_sh.at[idx_v.at[j]], add=True
        )
        return carry

    lax.fori_loop(0, _KCH, body, 0)
    plsc.subcore_barrier()
    pltpu.sync_copy(
        acc_sh.at[pl.ds(s * _RPT, _RPT)], out_hbm.at[c].at[pl.ds(s * _RPT, _RPT)]
    )


def _proj_body(x_ref, wp_ref, bp_ref, o_ref):
    o_ref[...] = jax.nn.relu(
        jnp.dot(x_ref[...], wp_ref[...], preferred_element_type=jnp.float32)
        + bp_ref[...]
    )


def _edgenet_body(ea_ref, we1_ref, be1_ref, we2_ref, be2_ref, w_ref):
    z = jax.nn.relu(
        jnp.dot(ea_ref[...], we1_ref[...], preferred_element_type=jnp.float32)
        + be1_ref[...]
    )
    w_ref[...] = (
        jnp.dot(z, we2_ref[...], preferred_element_type=jnp.float32) + be2_ref[...]
    )


def _msg_body(w_ref, hs_ref, o_ref):
    g = pl.program_id(0)
    hs = hs_ref[...]
    w = w_ref[...]
    acc = hs[:, 0:1] * w[:, 0:H]
    for i in range(1, H):
        acc = acc + hs[:, i : i + 1] * w[:, i * H : (i + 1) * H]
    row = g * _ET + lax.broadcasted_iota(jnp.int32, (_ET, H), 0)
    o_ref[...] = jnp.where(row < E, acc, 0.0)


def _gru_body(p_ref, hid_ref, bnn_ref, wihT_ref, whhT_ref, bih_ref, bhh_ref, o_ref):
    agg = p_ref[0] + p_ref[1]
    h_new = jax.nn.relu(agg + bnn_ref[...])
    hidden = hid_ref[...]
    gi = jnp.dot(h_new, wihT_ref[...], preferred_element_type=jnp.float32) + bih_ref[...]
    gh = jnp.dot(hidden, whhT_ref[...], preferred_element_type=jnp.float32) + bhh_ref[...]
    i_r, i_z, i_n = gi[:, 0:H], gi[:, H : 2 * H], gi[:, 2 * H : 3 * H]
    h_r, h_z, h_n = gh[:, 0:H], gh[:, H : 2 * H], gh[:, 2 * H : 3 * H]
    r = jax.nn.sigmoid(i_r + h_r)
    z = jax.nn.sigmoid(i_z + h_z)
    n = jnp.tanh(i_n + r * h_n)
    o_ref[...] = (1.0 - z) * n + z * hidden


def _s2s_body(
    h_ref, wih0T_ref, wih12T_ref, whhT_ref, bih_ref, bhh_ref,
    wp1_ref, bp1_ref, wp2_ref, bp2_ref, o_ref,
):
    h_nodes = h_ref[...]
    f32 = jnp.float32
    q_star = jnp.zeros((1, 2 * H), dtype=f32)
    hs = [jnp.zeros((1, H), dtype=f32) for _ in range(NLAYER_S2S)]
    cs = [jnp.zeros((1, H), dtype=f32) for _ in range(NLAYER_S2S)]
    for _ in range(NSTEP_S2S):
        inp_l = q_star
        for l in range(NLAYER_S2S):
            wT = wih0T_ref[...] if l == 0 else wih12T_ref[l - 1]
            g = (
                jnp.dot(inp_l, wT, preferred_element_type=f32)
                + bih_ref[l]
                + jnp.dot(hs[l], whhT_ref[l], preferred_element_type=f32)
                + bhh_ref[l]
            )
            i = jax.nn.sigmoid(g[:, 0:H])
            f = jax.nn.sigmoid(g[:, H : 2 * H])
            gg = jnp.tanh(g[:, 2 * H : 3 * H])
            o = jax.nn.sigmoid(g[:, 3 * H : 4 * H])
            cs[l] = f * cs[l] + i * gg
            hs[l] = o * jnp.tanh(cs[l])
            inp_l = hs[l]
        q = inp_l
        e = jnp.sum(h_nodes * q, axis=-1, keepdims=True)
        m = jnp.max(e)
        p = jnp.exp(e - m)
        alpha = p / jnp.sum(p)
        readout = jnp.sum(alpha * h_nodes, axis=0, keepdims=True)
        q_star = jnp.concatenate([q, readout], axis=1)
    hid = jax.nn.relu(
        jnp.dot(q_star, wp1_ref[...], preferred_element_type=f32) + bp1_ref[...]
    )
    o_ref[...] = jnp.dot(hid, wp2_ref[...], preferred_element_type=f32) + bp2_ref[...]


def _full(shape):
    nd = len(shape)
    return pl.BlockSpec(shape, lambda g: (0,) * nd)


def kernel(x, edge_index, edge_attr, Wp, bp, We1, be1, We2, be2, b_nn, gru_Wih, gru_Whh, gru_bih, gru_bhh, lstm_Wih0, lstm_Wih12, lstm_Whh, lstm_bih, lstm_bhh, Wp1, bp1, Wp2, bp2):
    f32 = jnp.float32
    src = edge_index[0]
    dst = edge_index[1]
    pad = _EPAD - E
    spread = (jnp.arange(pad, dtype=jnp.int32) * 73) % N
    src_p = jnp.concatenate([src, spread]).reshape(_NW * _KCH, _CHUNK)
    src_p = src_p.reshape(_NW, _KCH, _CHUNK)
    dst_p = jnp.concatenate([dst, spread]).reshape(_NW, _KCH, _CHUNK)
    ea_p = jnp.concatenate([edge_attr, jnp.zeros((pad, D_E), f32)])
    zeros_n = jnp.zeros((N, H), f32)

    h0 = pl.pallas_call(
        _proj_body,
        grid=(1,),
        in_specs=[_full((N, D_IN)), _full((D_IN, H)), _full((1, H))],
        out_specs=_full((N, H)),
        out_shape=jax.ShapeDtypeStruct((N, H), f32),
    )(x, Wp, bp.reshape(1, H))

    w_edge = pl.pallas_call(
        _edgenet_body,
        grid=(_NT_E,),
        in_specs=[
            pl.BlockSpec((_ET, D_E), lambda g: (g, 0)),
            _full((D_E, EH)),
            _full((1, EH)),
            _full((EH, H * H)),
            _full((1, H * H)),
        ],
        out_specs=pl.BlockSpec((_ET, H * H), lambda g: (g, 0)),
        out_shape=jax.ShapeDtypeStruct((_EPAD, H * H), f32),
    )(ea_p, We1, be1.reshape(1, EH), We2, be2.reshape(1, H * H))

    gru_args = (
        b_nn.reshape(1, H),
        gru_Wih.T,
        gru_Whh.T,
        gru_bih.reshape(1, 3 * H),
        gru_bhh.reshape(1, 3 * H),
    )

    hidden = h0
    h_nodes = h0
    for _ in range(NSTEP_MP):
        hs_e = _sc_gather(h_nodes, src_p)
        msg = pl.pallas_call(
            _msg_body,
            grid=(_NT_E,),
            in_specs=[
                pl.BlockSpec((_ET, H * H), lambda g: (g, 0)),
                pl.BlockSpec((_ET, H), lambda g: (g, 0)),
            ],
            out_specs=pl.BlockSpec((_ET, H), lambda g: (g, 0)),
            out_shape=jax.ShapeDtypeStruct((_EPAD, H), f32),
        )(w_edge, hs_e)
        parts = _sc_scatter_add(msg, dst_p, zeros_n)
        hidden = pl.pallas_call(
            _gru_body,
            grid=(1,),
            in_specs=[
                _full((_NC, N, H)),
                _full((N, H)),
                _full((1, H)),
                _full((H, 3 * H)),
                _full((H, 3 * H)),
                _full((1, 3 * H)),
                _full((1, 3 * H)),
            ],
            out_specs=_full((N, H)),
            out_shape=jax.ShapeDtypeStruct((N, H), f32),
        )(parts, hidden, *gru_args)
        h_nodes = hidden

    out = pl.pallas_call(
        _s2s_body,
        grid=(1,),
        in_specs=[
            _full((N, H)),
            _full((2 * H, 4 * H)),
            _full((NLAYER_S2S - 1, H, 4 * H)),
            _full((NLAYER_S2S, H, 4 * H)),
            _full((NLAYER_S2S, 1, 4 * H)),
            _full((NLAYER_S2S, 1, 4 * H)),
            _full((2 * H, H)),
            _full((1, H)),
            _full((H, 1)),
            _full((1, 1)),
        ],
        out_specs=_full((1, 1)),
        out_shape=jax.ShapeDtypeStruct((1, 1), f32),
    )(
        h_nodes,
        lstm_Wih0.T,
        jnp.transpose(lstm_Wih12, (0, 2, 1)),
        jnp.transpose(lstm_Whh, (0, 2, 1)),
        lstm_bih.reshape(NLAYER_S2S, 1, 4 * H),
        lstm_bhh.reshape(NLAYER_S2S, 1, 4 * H),
        Wp1,
        bp1.reshape(1, H),
        Wp2,
        bp2.reshape(1, 1),
    )
    return out


# speedup vs baseline: 3.1817x; 2.9139x over previous
import functools

import jax
import jax.numpy as jnp
from jax import lax
from jax.experimental import pallas as pl
from jax.experimental.pallas import tpu as pltpu
from jax.experimental.pallas import tpu_sc as plsc

N = 10000
E = 160000
D_IN = 128
D_E = 16
H = 16
EH = 128
NSTEP_MP = 3
NSTEP_S2S = 6
NLAYER_S2S = 3

_NC = 2
_NS = 16
_NW = _NC * _NS

_CHUNK = 100
_KCH = 50
_BPW = _CHUNK * _KCH
_ET = 4000
_NT_E = E // _ET
_RPT = N // _NS

_sc_mesh = plsc.VectorSubcoreMesh(core_axis_name="c", subcore_axis_name="s")
_sc_params = pltpu.CompilerParams(use_tc_tiling_on_sc=False)


@functools.partial(
    pl.kernel,
    out_type=jax.ShapeDtypeStruct((E, H), jnp.float32),
    mesh=_sc_mesh,
    compiler_params=_sc_params,
    scratch_types=[
        pltpu.VMEM((_KCH, _CHUNK), jnp.int32),
        pltpu.VMEM((_BPW, H), jnp.float32),
        pltpu.SemaphoreType.DMA,
    ],
)
def _sc_gather(table_hbm, idx_hbm, out_hbm, idx_v, rows_v, sem):
    wid = lax.axis_index("s") * _NC + lax.axis_index("c")
    pltpu.sync_copy(idx_hbm.at[wid], idx_v)

    def grp(g, carry):
        for b in range(10):
            j = g * 10 + b
            pltpu.async_copy(
                table_hbm.at[idx_v.at[j]], rows_v.at[pl.ds(j * _CHUNK, _CHUNK)], sem
            )
        for b in range(10):
            j = g * 10 + b
            pltpu.make_async_copy(
                table_hbm.at[idx_v.at[j]], rows_v.at[pl.ds(j * _CHUNK, _CHUNK)], sem
            ).wait()
        return carry

    lax.fori_loop(0, _KCH // 10, grp, 0)
    pltpu.sync_copy(rows_v, out_hbm.at[pl.ds(wid * _BPW, _BPW)])


@functools.partial(
    pl.kernel,
    out_type=jax.ShapeDtypeStruct((_NC, N, H), jnp.float32),
    mesh=_sc_mesh,
    compiler_params=_sc_params,
    scratch_types=[
        pltpu.VMEM((_KCH, _CHUNK), jnp.int32),
        pltpu.VMEM((_BPW, H), jnp.float32),
        pltpu.VMEM_SHARED((N, H), jnp.float32),
    ],
)
def _sc_scatter_add(msg_hbm, dst_hbm, zeros_hbm, out_hbm, idx_v, msg_v, acc_sh):
    c = lax.axis_index("c")
    s = lax.axis_index("s")
    wid = s * _NC + c
    pltpu.sync_copy(zeros_hbm.at[pl.ds(s * _RPT, _RPT)], acc_sh.at[pl.ds(s * _RPT, _RPT)])
    pltpu.sync_copy(dst_hbm.at[wid], idx_v)
    pltpu.sync_copy(msg_hbm.at[pl.ds(wid * _BPW, _BPW)], msg_v)
    plsc.subcore_barrier()

    def body(j, carry):
        pltpu.sync_copy(
            msg_v.at[pl.ds(j * _CHUNK, _CHUNK)], acc_sh.at[idx_v.at[j]], add=True
        )
        return carry

    lax.fori_loop(0, _KCH, body, 0)
    plsc.subcore_barrier()
    pltpu.sync_copy(
        acc_sh.at[pl.ds(s * _RPT, _RPT)], out_hbm.at[c].at[pl.ds(s * _RPT, _RPT)]
    )


def _proj_body(x_ref, wp_ref, bp_ref, o_ref):
    o_ref[...] = jax.nn.relu(
        jnp.dot(x_ref[...], wp_ref[...], preferred_element_type=jnp.float32)
        + bp_ref[...]
    )


def _edgenet_body(ea_ref, we1_ref, be1_ref, we2_ref, be2_ref, w_ref):
    z = jax.nn.relu(
        jnp.dot(ea_ref[...], we1_ref[...], preferred_element_type=jnp.float32)
        + be1_ref[...]
    )
    w_ref[...] = (
        jnp.dot(z, we2_ref[...], preferred_element_type=jnp.float32) + be2_ref[...]
    )


def _msg_body(w_ref, hs_ref, r_ref, s_ref, o_ref):
    hs = hs_ref[...]
    w = w_ref[...]
    hs_t = jnp.dot(hs, r_ref[...], preferred_element_type=jnp.float32)
    o_ref[...] = jnp.dot(hs_t * w, s_ref[...], preferred_element_type=jnp.float32)


def _gru_body(p_ref, hid_ref, bnn_ref, wihT_ref, whhT_ref, bih_ref, bhh_ref, o_ref):
    agg = p_ref[0] + p_ref[1]
    h_new = jax.nn.relu(agg + bnn_ref[...])
    hidden = hid_ref[...]
    gi = jnp.dot(h_new, wihT_ref[...], preferred_element_type=jnp.float32) + bih_ref[...]
    gh = jnp.dot(hidden, whhT_ref[...], preferred_element_type=jnp.float32) + bhh_ref[...]
    i_r, i_z, i_n = gi[:, 0:H], gi[:, H : 2 * H], gi[:, 2 * H : 3 * H]
    h_r, h_z, h_n = gh[:, 0:H], gh[:, H : 2 * H], gh[:, 2 * H : 3 * H]
    r = jax.nn.sigmoid(i_r + h_r)
    z = jax.nn.sigmoid(i_z + h_z)
    n = jnp.tanh(i_n + r * h_n)
    o_ref[...] = (1.0 - z) * n + z * hidden


def _s2s_body(
    h_ref, wih0T_ref, wih12T_ref, whhT_ref, bih_ref, bhh_ref,
    wp1_ref, bp1_ref, wp2_ref, bp2_ref, o_ref,
):
    h_nodes = h_ref[...]
    f32 = jnp.float32
    q_star = jnp.zeros((1, 2 * H), dtype=f32)
    hs = [jnp.zeros((1, H), dtype=f32) for _ in range(NLAYER_S2S)]
    cs = [jnp.zeros((1, H), dtype=f32) for _ in range(NLAYER_S2S)]
    for _ in range(NSTEP_S2S):
        inp_l = q_star
        for l in range(NLAYER_S2S):
            wT = wih0T_ref[...] if l == 0 else wih12T_ref[l - 1]
            g = (
                jnp.dot(inp_l, wT, preferred_element_type=f32)
                + bih_ref[l]
                + jnp.dot(hs[l], whhT_ref[l], preferred_element_type=f32)
                + bhh_ref[l]
            )
            i = jax.nn.sigmoid(g[:, 0:H])
            f = jax.nn.sigmoid(g[:, H : 2 * H])
            gg = jnp.tanh(g[:, 2 * H : 3 * H])
            o = jax.nn.sigmoid(g[:, 3 * H : 4 * H])
            cs[l] = f * cs[l] + i * gg
            hs[l] = o * jnp.tanh(cs[l])
            inp_l = hs[l]
        q = inp_l
        e = jnp.sum(h_nodes * q, axis=-1, keepdims=True)
        m = jnp.max(e)
        p = jnp.exp(e - m)
        alpha = p / jnp.sum(p)
        readout = jnp.sum(alpha * h_nodes, axis=0, keepdims=True)
        q_star = jnp.concatenate([q, readout], axis=1)
    hid = jax.nn.relu(
        jnp.dot(q_star, wp1_ref[...], preferred_element_type=f32) + bp1_ref[...]
    )
    o_ref[...] = jnp.dot(hid, wp2_ref[...], preferred_element_type=f32) + bp2_ref[...]


def _full(shape):
    nd = len(shape)
    return pl.BlockSpec(shape, lambda g: (0,) * nd)


def kernel(x, edge_index, edge_attr, Wp, bp, We1, be1, We2, be2, b_nn, gru_Wih, gru_Whh, gru_bih, gru_bhh, lstm_Wih0, lstm_Wih12, lstm_Whh, lstm_bih, lstm_bhh, Wp1, bp1, Wp2, bp2):
    f32 = jnp.float32
    src_p = edge_index[0].reshape(_NW, _KCH, _CHUNK)
    dst_p = edge_index[1].reshape(_NW, _KCH, _CHUNK)
    zeros_n = jnp.zeros((N, H), f32)
    sel = jnp.repeat(jnp.eye(H, dtype=f32), H, axis=0)
    rep = jnp.tile(jnp.eye(H, dtype=f32), (1, H))
    We2p = We2.reshape(EH, H, H).swapaxes(1, 2).reshape(EH, H * H)
    be2p = be2.reshape(H, H).T.reshape(H * H)

    h0 = pl.pallas_call(
        _proj_body,
        grid=(1,),
        in_specs=[_full((N, D_IN)), _full((D_IN, H)), _full((1, H))],
        out_specs=_full((N, H)),
        out_shape=jax.ShapeDtypeStruct((N, H), f32),
    )(x, Wp, bp.reshape(1, H))

    w_edge = pl.pallas_call(
        _edgenet_body,
        grid=(_NT_E,),
        in_specs=[
            pl.BlockSpec((_ET, D_E), lambda g: (g, 0)),
            _full((D_E, EH)),
            _full((1, EH)),
            _full((EH, H * H)),
            _full((1, H * H)),
        ],
        out_specs=pl.BlockSpec((_ET, H * H), lambda g: (g, 0)),
        out_shape=jax.ShapeDtypeStruct((E, H * H), f32),
    )(edge_attr, We1, be1.reshape(1, EH), We2p, be2p.reshape(1, H * H))

    gru_args = (
        b_nn.reshape(1, H),
        gru_Wih.T,
        gru_Whh.T,
        gru_bih.reshape(1, 3 * H),
        gru_bhh.reshape(1, 3 * H),
    )

    hidden = h0
    h_nodes = h0
    for _ in range(NSTEP_MP):
        hs_e = _sc_gather(h_nodes, src_p)
        msg = pl.pallas_call(
            _msg_body,
            grid=(_NT_E,),
            in_specs=[
                pl.BlockSpec((_ET, H * H), lambda g: (g, 0)),
                pl.BlockSpec((_ET, H), lambda g: (g, 0)),
                _full((H, H * H)),
                _full((H * H, H)),
            ],
            out_specs=pl.BlockSpec((_ET, H), lambda g: (g, 0)),
            out_shape=jax.ShapeDtypeStruct((E, H), f32),
        )(w_edge, hs_e, rep, sel)
        parts = _sc_scatter_add(msg, dst_p, zeros_n)
        hidden = pl.pallas_call(
            _gru_body,
            grid=(1,),
            in_specs=[
                _full((_NC, N, H)),
                _full((N, H)),
                _full((1, H)),
                _full((H, 3 * H)),
                _full((H, 3 * H)),
                _full((1, 3 * H)),
                _full((1, 3 * H)),
            ],
            out_specs=_full((N, H)),
            out_shape=jax.ShapeDtypeStruct((N, H), f32),
        )(parts, hidden, *gru_args)
        h_nodes = hidden

    out = pl.pallas_call(
        _s2s_body,
        grid=(1,),
        in_specs=[
            _full((N, H)),
            _full((2 * H, 4 * H)),
            _full((NLAYER_S2S - 1, H, 4 * H)),
            _full((NLAYER_S2S, H, 4 * H)),
            _full((NLAYER_S2S, 1, 4 * H)),
            _full((NLAYER_S2S, 1, 4 * H)),
            _full((2 * H, H)),
            _full((1, H)),
            _full((H, 1)),
            _full((1, 1)),
        ],
        out_specs=_full((1, 1)),
        out_shape=jax.ShapeDtypeStruct((1, 1), f32),
    )(
        h_nodes,
        lstm_Wih0.T,
        jnp.transpose(lstm_Wih12, (0, 2, 1)),
        jnp.transpose(lstm_Whh, (0, 2, 1)),
        lstm_bih.reshape(NLAYER_S2S, 1, 4 * H),
        lstm_bhh.reshape(NLAYER_S2S, 1, 4 * H),
        Wp1,
        bp1.reshape(1, H),
        Wp2,
        bp2.reshape(1, 1),
    )
    return out


# speedup vs baseline: 5.1113x; 1.6064x over previous
import functools

import jax
import jax.numpy as jnp
from jax import lax
from jax.experimental import pallas as pl
from jax.experimental.pallas import tpu as pltpu
from jax.experimental.pallas import tpu_sc as plsc

N = 10000
E = 160000
D_IN = 128
D_E = 16
H = 16
EH = 128
NSTEP_MP = 3
NSTEP_S2S = 6
NLAYER_S2S = 3

_NC = 2
_NS = 16
_NW = _NC * _NS

_CHUNK = 100
_KCH = 50
_BPW = _CHUNK * _KCH
_ET = 3200
_NT_E = E // _ET
_RPT = N // _NS
_NP = N // 8
_ETP = _ET // 8

_sc_mesh = plsc.VectorSubcoreMesh(core_axis_name="c", subcore_axis_name="s")
_sc_params = pltpu.CompilerParams(use_tc_tiling_on_sc=False)


@functools.partial(
    pl.kernel,
    out_type=jax.ShapeDtypeStruct((E, H), jnp.float32),
    mesh=_sc_mesh,
    compiler_params=_sc_params,
    scratch_types=[
        pltpu.VMEM((_KCH, _CHUNK), jnp.int32),
        pltpu.VMEM((_BPW, H), jnp.float32),
        pltpu.SemaphoreType.DMA,
    ],
)
def _sc_gather(table_hbm, idx_hbm, out_hbm, idx_v, rows_v, sem):
    wid = lax.axis_index("s") * _NC + lax.axis_index("c")
    pltpu.sync_copy(idx_hbm.at[wid], idx_v)

    def grp(g, carry):
        for b in range(10):
            j = g * 10 + b
            pltpu.async_copy(
                table_hbm.at[idx_v.at[j]], rows_v.at[pl.ds(j * _CHUNK, _CHUNK)], sem
            )
        for b in range(10):
            j = g * 10 + b
            pltpu.make_async_copy(
                table_hbm.at[idx_v.at[j]], rows_v.at[pl.ds(j * _CHUNK, _CHUNK)], sem
            ).wait()
        return carry

    lax.fori_loop(0, _KCH // 10, grp, 0)
    pltpu.sync_copy(rows_v, out_hbm.at[pl.ds(wid * _BPW, _BPW)])


@functools.partial(
    pl.kernel,
    out_type=jax.ShapeDtypeStruct((_NC, N, H), jnp.float32),
    mesh=_sc_mesh,
    compiler_params=_sc_params,
    scratch_types=[
        pltpu.VMEM((_KCH, _CHUNK), jnp.int32),
        pltpu.VMEM((_BPW, H), jnp.float32),
        pltpu.VMEM_SHARED((N, H), jnp.float32),
    ],
)
def _sc_scatter_add(msg_hbm, dst_hbm, zeros_hbm, out_hbm, idx_v, msg_v, acc_sh):
    c = lax.axis_index("c")
    s = lax.axis_index("s")
    wid = s * _NC + c
    pltpu.sync_copy(zeros_hbm.at[pl.ds(s * _RPT, _RPT)], acc_sh.at[pl.ds(s * _RPT, _RPT)])
    pltpu.sync_copy(dst_hbm.at[wid], idx_v)
    pltpu.sync_copy(msg_hbm.at[pl.ds(wid * _BPW, _BPW)], msg_v)
    plsc.subcore_barrier()

    def body(j, carry):
        pltpu.sync_copy(
            msg_v.at[pl.ds(j * _CHUNK, _CHUNK)], acc_sh.at[idx_v.at[j]], add=True
        )
        return carry

    lax.fori_loop(0, _KCH, body, 0)
    plsc.subcore_barrier()
    pltpu.sync_copy(
        acc_sh.at[pl.ds(s * _RPT, _RPT)], out_hbm.at[c].at[pl.ds(s * _RPT, _RPT)]
    )


def _proj_body(x_ref, wp_ref, bp_ref, o_ref):
    o_ref[...] = jax.nn.relu(
        jnp.dot(x_ref[...], wp_ref[...], preferred_element_type=jnp.float32)
        + bp_ref[...]
    )


def _edgenet_body(ea_ref, we1_ref, be1_ref, we2_ref, be2_ref, w_ref):
    ea = ea_ref[...]
    for m in range(8):
        ea_m = ea[:, m * H : (m + 1) * H]
        z = jax.nn.relu(
            jnp.dot(ea_m, we1_ref[...], preferred_element_type=jnp.float32)
            + be1_ref[...]
        )
        w_ref[m] = (
            jnp.dot(z, we2_ref[...], preferred_element_type=jnp.float32)
            + be2_ref[...]
        )


def _msg_body(w_ref, hs_ref, r_ref, s_ref, o_ref):
    hs = hs_ref[...]
    outs = []
    for m in range(8):
        hs_m = hs[:, m * H : (m + 1) * H]
        hs_t = jnp.dot(hs_m, r_ref[...], preferred_element_type=jnp.float32)
        outs.append(
            jnp.dot(hs_t * w_ref[m], s_ref[...], preferred_element_type=jnp.float32)
        )
    o_ref[...] = jnp.concatenate(outs, axis=1)


def _gru_body(p_ref, hid_ref, bnn_ref, wihT_ref, whhT_ref, bih_ref, bhh_ref, o_ref):
    agg = p_ref[0] + p_ref[1]
    h_new = jax.nn.relu(agg + bnn_ref[...])
    hidden = hid_ref[...]
    gi = jnp.dot(h_new, wihT_ref[...], preferred_element_type=jnp.float32) + bih_ref[...]
    gh = jnp.dot(hidden, whhT_ref[...], preferred_element_type=jnp.float32) + bhh_ref[...]
    i_r, i_z, i_n = gi[:, 0:H], gi[:, H : 2 * H], gi[:, 2 * H : 3 * H]
    h_r, h_z, h_n = gh[:, 0:H], gh[:, H : 2 * H], gh[:, 2 * H : 3 * H]
    r = jax.nn.sigmoid(i_r + h_r)
    z = jax.nn.sigmoid(i_z + h_z)
    n = jnp.tanh(i_n + r * h_n)
    o_ref[...] = (1.0 - z) * n + z * hidden


def _s2s_body(
    h_ref, rep8_ref, bs_ref, fold_ref,
    wih0T_ref, wih12T_ref, whhT_ref, bih_ref, bhh_ref,
    wp1_ref, bp1_ref, wp2_ref, bp2_ref, o_ref,
):
    h_p = h_ref[...]
    f32 = jnp.float32
    q_star = jnp.zeros((1, 2 * H), dtype=f32)
    hs = [jnp.zeros((1, H), dtype=f32) for _ in range(NLAYER_S2S)]
    cs = [jnp.zeros((1, H), dtype=f32) for _ in range(NLAYER_S2S)]
    for _ in range(NSTEP_S2S):
        inp_l = q_star
        for l in range(NLAYER_S2S):
            wT = wih0T_ref[...] if l == 0 else wih12T_ref[l - 1]
            g = (
                jnp.dot(inp_l, wT, preferred_element_type=f32)
                + bih_ref[l]
                + jnp.dot(hs[l], whhT_ref[l], preferred_element_type=f32)
                + bhh_ref[l]
            )
            i = jax.nn.sigmoid(g[:, 0:H])
            f = jax.nn.sigmoid(g[:, H : 2 * H])
            gg = jnp.tanh(g[:, 2 * H : 3 * H])
            o = jax.nn.sigmoid(g[:, 3 * H : 4 * H])
            cs[l] = f * cs[l] + i * gg
            hs[l] = o * jnp.tanh(cs[l])
            inp_l = hs[l]
        q = inp_l
        qt = jnp.dot(q, rep8_ref[...], preferred_element_type=f32)
        ep = h_p * qt
        er = jnp.dot(ep, bs_ref[...], preferred_element_type=f32)
        m = jnp.max(er)
        p = jnp.exp(er - m)
        alpha = p / (jnp.sum(p) * (1.0 / H))
        ro_p = jnp.sum(alpha * h_p, axis=0, keepdims=True)
        readout = jnp.dot(ro_p, fold_ref[...], preferred_element_type=f32)
        q_star = jnp.concatenate([q, readout], axis=1)
    hid = jax.nn.relu(
        jnp.dot(q_star, wp1_ref[...], preferred_element_type=f32) + bp1_ref[...]
    )
    o_ref[...] = jnp.dot(hid, wp2_ref[...], preferred_element_type=f32) + bp2_ref[...]


def _full(shape):
    nd = len(shape)
    return pl.BlockSpec(shape, lambda g: (0,) * nd)


def kernel(x, edge_index, edge_attr, Wp, bp, We1, be1, We2, be2, b_nn, gru_Wih, gru_Whh, gru_bih, gru_bhh, lstm_Wih0, lstm_Wih12, lstm_Whh, lstm_bih, lstm_bhh, Wp1, bp1, Wp2, bp2):
    f32 = jnp.float32
    src_p = edge_index[0].reshape(_NW, _KCH, _CHUNK)
    dst_p = edge_index[1].reshape(_NW, _KCH, _CHUNK)
    zeros_n = jnp.zeros((N, H), f32)
    ea_pk = edge_attr.reshape(E // 8, 128)

    eye8 = jnp.eye(8, dtype=f32)
    rep = jnp.tile(jnp.eye(H, dtype=f32), (1, H))
    sel = jnp.repeat(jnp.eye(H, dtype=f32), H, axis=0)
    We2p = We2.reshape(EH, H, H).swapaxes(1, 2).reshape(EH, H * H)
    be2p = be2.reshape(H, H).T.reshape(H * H)

    rep8 = jnp.tile(jnp.eye(H, dtype=f32), (1, 8))
    bs = jnp.kron(eye8, jnp.ones((H, H), dtype=f32))
    fold = jnp.tile(jnp.eye(H, dtype=f32), (8, 1))

    gru_args = (
        b_nn.reshape(1, H),
        gru_Wih.T,
        gru_Whh.T,
        gru_bih.reshape(1, 3 * H),
        gru_bhh.reshape(1, 3 * H),
    )

    h0 = pl.pallas_call(
        _proj_body,
        grid=(1,),
        in_specs=[_full((N, D_IN)), _full((D_IN, H)), _full((1, H))],
        out_specs=_full((N, H)),
        out_shape=jax.ShapeDtypeStruct((N, H), f32),
    )(x, Wp, bp.reshape(1, H))

    w_edge = pl.pallas_call(
        _edgenet_body,
        grid=(_NT_E,),
        in_specs=[
            pl.BlockSpec((_ETP, 128), lambda g: (g, 0)),
            _full((D_E, EH)),
            _full((1, EH)),
            _full((EH, H * H)),
            _full((1, H * H)),
        ],
        out_specs=pl.BlockSpec((8, _ETP, H * H), lambda g: (0, g, 0)),
        out_shape=jax.ShapeDtypeStruct((8, E // 8, H * H), f32),
    )(ea_pk, We1, be1.reshape(1, EH), We2p, be2p.reshape(1, H * H))

    hidden = h0
    h_nodes = h0
    for _ in range(NSTEP_MP):
        hs_e = _sc_gather(h_nodes, src_p)
        msg = pl.pallas_call(
            _msg_body,
            grid=(_NT_E,),
            in_specs=[
                pl.BlockSpec((8, _ETP, H * H), lambda g: (0, g, 0)),
                pl.BlockSpec((_ETP, 128), lambda g: (g, 0)),
                _full((H, H * H)),
                _full((H * H, H)),
            ],
            out_specs=pl.BlockSpec((_ETP, 128), lambda g: (g, 0)),
            out_shape=jax.ShapeDtypeStruct((E // 8, 128), f32),
        )(w_edge, hs_e.reshape(E // 8, 128), rep, sel)
        parts = _sc_scatter_add(msg.reshape(E, H), dst_p, zeros_n)
        hidden = pl.pallas_call(
            _gru_body,
            grid=(1,),
            in_specs=[
                _full((_NC, N, H)),
                _full((N, H)),
                _full((1, H)),
                _full((H, 3 * H)),
                _full((H, 3 * H)),
                _full((1, 3 * H)),
                _full((1, 3 * H)),
            ],
            out_specs=_full((N, H)),
            out_shape=jax.ShapeDtypeStruct((N, H), f32),
        )(parts, hidden, *gru_args)
        h_nodes = hidden

    out = pl.pallas_call(
        _s2s_body,
        grid=(1,),
        in_specs=[
            _full((_NP, 128)),
            _full((H, 128)),
            _full((128, 128)),
            _full((128, H)),
            _full((2 * H, 4 * H)),
            _full((NLAYER_S2S - 1, H, 4 * H)),
            _full((NLAYER_S2S, H, 4 * H)),
            _full((NLAYER_S2S, 1, 4 * H)),
            _full((NLAYER_S2S, 1, 4 * H)),
            _full((2 * H, H)),
            _full((1, H)),
            _full((H, 1)),
            _full((1, 1)),
        ],
        out_specs=_full((1, 1)),
        out_shape=jax.ShapeDtypeStruct((1, 1), f32),
    )(
        h_nodes.reshape(_NP, 128),
        rep8,
        bs,
        fold,
        lstm_Wih0.T,
        jnp.transpose(lstm_Wih12, (0, 2, 1)),
        jnp.transpose(lstm_Whh, (0, 2, 1)),
        lstm_bih.reshape(NLAYER_S2S, 1, 4 * H),
        lstm_bhh.reshape(NLAYER_S2S, 1, 4 * H),
        Wp1,
        bp1.reshape(1, H),
        Wp2,
        bp2.reshape(1, 1),
    )
    return out


# speedup vs baseline: 6.2628x; 1.2253x over previous
import functools

import jax
import jax.numpy as jnp
from jax import lax
from jax.experimental import pallas as pl
from jax.experimental.pallas import tpu as pltpu
from jax.experimental.pallas import tpu_sc as plsc

N = 10000
E = 160000
D_IN = 128
D_E = 16
H = 16
EH = 128
NSTEP_MP = 3
NSTEP_S2S = 6
NLAYER_S2S = 3

_NC = 2
_NS = 16
_NW = _NC * _NS

_CHUNK = 100
_KCH = 50
_BPW = _CHUNK * _KCH
_ET = 3200
_NT_E = E // _ET
_RPT = N // _NS
_NP = N // 8
_ETP = _ET // 8

_sc_mesh = plsc.VectorSubcoreMesh(core_axis_name="c", subcore_axis_name="s")
_sc_params = pltpu.CompilerParams(use_tc_tiling_on_sc=False)


@functools.partial(
    pl.kernel,
    out_type=jax.ShapeDtypeStruct((E, H), jnp.float32),
    mesh=_sc_mesh,
    compiler_params=_sc_params,
    scratch_types=[
        pltpu.VMEM((_KCH, _CHUNK), jnp.int32),
        pltpu.VMEM((_BPW, H), jnp.float32),
        pltpu.SemaphoreType.DMA,
    ],
)
def _sc_gather(table_hbm, idx_hbm, out_hbm, idx_v, rows_v, sem):
    wid = lax.axis_index("s") * _NC + lax.axis_index("c")
    pltpu.sync_copy(idx_hbm.at[wid], idx_v)

    def grp(g, carry):
        for b in range(10):
            j = g * 10 + b
            pltpu.async_copy(
                table_hbm.at[idx_v.at[j]], rows_v.at[pl.ds(j * _CHUNK, _CHUNK)], sem
            )
        for b in range(10):
            j = g * 10 + b
            pltpu.make_async_copy(
                table_hbm.at[idx_v.at[j]], rows_v.at[pl.ds(j * _CHUNK, _CHUNK)], sem
            ).wait()
        return carry

    lax.fori_loop(0, _KCH // 10, grp, 0)
    pltpu.sync_copy(rows_v, out_hbm.at[pl.ds(wid * _BPW, _BPW)])


@functools.partial(
    pl.kernel,
    out_type=jax.ShapeDtypeStruct((_NC, N, H), jnp.float32),
    mesh=_sc_mesh,
    compiler_params=_sc_params,
    scratch_types=[
        pltpu.VMEM((_KCH, _CHUNK), jnp.int32),
        pltpu.VMEM((_BPW, H), jnp.float32),
        pltpu.VMEM_SHARED((N, H), jnp.float32),
    ],
)
def _sc_scatter_add(msg_hbm, dst_hbm, zeros_hbm, out_hbm, idx_v, msg_v, acc_sh):
    c = lax.axis_index("c")
    s = lax.axis_index("s")
    wid = s * _NC + c
    pltpu.sync_copy(zeros_hbm.at[pl.ds(s * _RPT, _RPT)], acc_sh.at[pl.ds(s * _RPT, _RPT)])
    pltpu.sync_copy(dst_hbm.at[wid], idx_v)
    pltpu.sync_copy(msg_hbm.at[pl.ds(wid * _BPW, _BPW)], msg_v)
    plsc.subcore_barrier()

    def body(j, carry):
        pltpu.sync_copy(
            msg_v.at[pl.ds(j * _CHUNK, _CHUNK)], acc_sh.at[idx_v.at[j]], add=True
        )
        return carry

    lax.fori_loop(0, _KCH, body, 0)
    plsc.subcore_barrier()
    pltpu.sync_copy(
        acc_sh.at[pl.ds(s * _RPT, _RPT)], out_hbm.at[c].at[pl.ds(s * _RPT, _RPT)]
    )


def _proj_body(x_ref, wp_ref, bp_ref, o_ref):
    o_ref[...] = jax.nn.relu(
        jnp.dot(x_ref[...], wp_ref[...], preferred_element_type=jnp.float32)
        + bp_ref[...]
    )


def _edgenet_body(ea_ref, we1_ref, be1_ref, we2_ref, be2_ref, w_ref):
    ea = ea_ref[...]
    for m in range(8):
        ea_m = ea[:, m * H : (m + 1) * H]
        z = jax.nn.relu(
            jnp.dot(ea_m, we1_ref[...], preferred_element_type=jnp.float32)
            + be1_ref[...]
        )
        w_ref[m] = (
            jnp.dot(z, we2_ref[...], preferred_element_type=jnp.float32)
            + be2_ref[...]
        ).astype(jnp.bfloat16)


def _msg_body(w_ref, hs_ref, r_ref, s_ref, o_ref):
    hs = hs_ref[...]
    outs = []
    for m in range(8):
        hs_m = hs[:, m * H : (m + 1) * H]
        hs_t = jnp.dot(hs_m, r_ref[...], preferred_element_type=jnp.float32)
        w_m = w_ref[m].astype(jnp.float32)
        outs.append(
            jnp.dot(hs_t * w_m, s_ref[...], preferred_element_type=jnp.float32)
        )
    o_ref[...] = jnp.concatenate(outs, axis=1)


def _gru_body(
    p_ref, hid_ref, bnn_ref, wbih_ref, wbhh_ref, bih_ref, bhh_ref,
    gr_ref, gz_ref, gn_ref, o_ref,
):
    f32 = jnp.float32
    agg = p_ref[0] + p_ref[1]
    hn = jax.nn.relu(agg + bnn_ref[...])
    hid = hid_ref[...]
    gi = jnp.dot(hn, wbih_ref[...], preferred_element_type=f32) + bih_ref[...]
    gh = jnp.dot(hid, wbhh_ref[...], preferred_element_type=f32) + bhh_ref[...]
    s = gi + gh
    r = jax.nn.sigmoid(jnp.dot(s, gr_ref[...], preferred_element_type=f32))
    z = jax.nn.sigmoid(jnp.dot(s, gz_ref[...], preferred_element_type=f32))
    i_n = jnp.dot(gi, gn_ref[...], preferred_element_type=f32)
    h_n = jnp.dot(gh, gn_ref[...], preferred_element_type=f32)
    n = jnp.tanh(i_n + r * h_n)
    o_ref[...] = (1.0 - z) * n + z * hid


def _s2s_body(
    h_ref, rep8_ref, bs_ref, fold_ref,
    wih0T_ref, wih12T_ref, whhT_ref, bih_ref, bhh_ref,
    wp1_ref, bp1_ref, wp2_ref, bp2_ref, o_ref,
):
    h_p = h_ref[...]
    f32 = jnp.float32
    q_star = jnp.zeros((1, 2 * H), dtype=f32)
    hs = [jnp.zeros((1, H), dtype=f32) for _ in range(NLAYER_S2S)]
    cs = [jnp.zeros((1, H), dtype=f32) for _ in range(NLAYER_S2S)]
    for _ in range(NSTEP_S2S):
        inp_l = q_star
        for l in range(NLAYER_S2S):
            wT = wih0T_ref[...] if l == 0 else wih12T_ref[l - 1]
            g = (
                jnp.dot(inp_l, wT, preferred_element_type=f32)
                + bih_ref[l]
                + jnp.dot(hs[l], whhT_ref[l], preferred_element_type=f32)
                + bhh_ref[l]
            )
            i = jax.nn.sigmoid(g[:, 0:H])
            f = jax.nn.sigmoid(g[:, H : 2 * H])
            gg = jnp.tanh(g[:, 2 * H : 3 * H])
            o = jax.nn.sigmoid(g[:, 3 * H : 4 * H])
            cs[l] = f * cs[l] + i * gg
            hs[l] = o * jnp.tanh(cs[l])
            inp_l = hs[l]
        q = inp_l
        qt = jnp.dot(q, rep8_ref[...], preferred_element_type=f32)
        ep = h_p * qt
        er = jnp.dot(ep, bs_ref[...], preferred_element_type=f32)
        m = jnp.max(er)
        p = jnp.exp(er - m)
        alpha = p / (jnp.sum(p) * (1.0 / H))
        ro_p = jnp.sum(alpha * h_p, axis=0, keepdims=True)
        readout = jnp.dot(ro_p, fold_ref[...], preferred_element_type=f32)
        q_star = jnp.concatenate([q, readout], axis=1)
    hid = jax.nn.relu(
        jnp.dot(q_star, wp1_ref[...], preferred_element_type=f32) + bp1_ref[...]
    )
    o_ref[...] = jnp.dot(hid, wp2_ref[...], preferred_element_type=f32) + bp2_ref[...]


def _full(shape):
    nd = len(shape)
    return pl.BlockSpec(shape, lambda g: (0,) * nd)


def kernel(x, edge_index, edge_attr, Wp, bp, We1, be1, We2, be2, b_nn, gru_Wih, gru_Whh, gru_bih, gru_bhh, lstm_Wih0, lstm_Wih12, lstm_Whh, lstm_bih, lstm_bhh, Wp1, bp1, Wp2, bp2):
    f32 = jnp.float32
    src_p = edge_index[0].reshape(_NW, _KCH, _CHUNK)
    dst_p = edge_index[1].reshape(_NW, _KCH, _CHUNK)
    zeros_n = jnp.zeros((N, H), f32)
    ea_pk = edge_attr.reshape(E // 8, 128)

    eye8 = jnp.eye(8, dtype=f32)
    rep = jnp.tile(jnp.eye(H, dtype=f32), (1, H))
    sel = jnp.repeat(jnp.eye(H, dtype=f32), H, axis=0)
    We2p = We2.reshape(EH, H, H).swapaxes(1, 2).reshape(EH, H * H)
    be2p = be2.reshape(H, H).T.reshape(H * H)

    rep8 = jnp.tile(jnp.eye(H, dtype=f32), (1, 8))
    bs = jnp.kron(eye8, jnp.ones((H, H), dtype=f32))
    fold = jnp.tile(jnp.eye(H, dtype=f32), (8, 1))

    gru_args = (
        jnp.tile(b_nn.reshape(1, H), (1, 8)),
        jnp.kron(eye8, gru_Wih.T),
        jnp.kron(eye8, gru_Whh.T),
        jnp.tile(gru_bih.reshape(1, 3 * H), (1, 8)),
        jnp.tile(gru_bhh.reshape(1, 3 * H), (1, 8)),
        jnp.kron(eye8, jnp.eye(3 * H, H, dtype=f32)),
        jnp.kron(eye8, jnp.eye(3 * H, H, k=-H, dtype=f32)),
        jnp.kron(eye8, jnp.eye(3 * H, H, k=-2 * H, dtype=f32)),
    )

    h0 = pl.pallas_call(
        _proj_body,
        grid=(1,),
        in_specs=[_full((N, D_IN)), _full((D_IN, H)), _full((1, H))],
        out_specs=_full((N, H)),
        out_shape=jax.ShapeDtypeStruct((N, H), f32),
    )(x, Wp, bp.reshape(1, H)).reshape(_NP, 128)

    w_edge = pl.pallas_call(
        _edgenet_body,
        grid=(_NT_E,),
        in_specs=[
            pl.BlockSpec((_ETP, 128), lambda g: (g, 0)),
            _full((D_E, EH)),
            _full((1, EH)),
            _full((EH, H * H)),
            _full((1, H * H)),
        ],
        out_specs=pl.BlockSpec((8, _ETP, H * H), lambda g: (0, g, 0)),
        out_shape=jax.ShapeDtypeStruct((8, E // 8, H * H), jnp.bfloat16),
    )(ea_pk, We1, be1.reshape(1, EH), We2p, be2p.reshape(1, H * H))

    hidden = h0
    h_nodes = h0
    for _ in range(NSTEP_MP):
        hs_e = _sc_gather(h_nodes.reshape(N, H), src_p)
        msg = pl.pallas_call(
            _msg_body,
            grid=(_NT_E,),
            in_specs=[
                pl.BlockSpec((8, _ETP, H * H), lambda g: (0, g, 0)),
                pl.BlockSpec((_ETP, 128), lambda g: (g, 0)),
                _full((H, H * H)),
                _full((H * H, H)),
            ],
            out_specs=pl.BlockSpec((_ETP, 128), lambda g: (g, 0)),
            out_shape=jax.ShapeDtypeStruct((E // 8, 128), f32),
        )(w_edge, hs_e.reshape(E // 8, 128), rep, sel)
        parts = _sc_scatter_add(msg.reshape(E, H), dst_p, zeros_n)
        hidden = pl.pallas_call(
            _gru_body,
            grid=(1,),
            in_specs=[
                _full((_NC, _NP, 128)),
                _full((_NP, 128)),
                _full((1, 128)),
                _full((128, 3 * H * 8)),
                _full((128, 3 * H * 8)),
                _full((1, 3 * H * 8)),
                _full((1, 3 * H * 8)),
                _full((3 * H * 8, 128)),
                _full((3 * H * 8, 128)),
                _full((3 * H * 8, 128)),
            ],
            out_specs=_full((_NP, 128)),
            out_shape=jax.ShapeDtypeStruct((_NP, 128), f32),
        )(parts.reshape(_NC, _NP, 128), hidden, *gru_args)
        h_nodes = hidden

    out = pl.pallas_call(
        _s2s_body,
        grid=(1,),
        in_specs=[
            _full((_NP, 128)),
            _full((H, 128)),
            _full((128, 128)),
            _full((128, H)),
            _full((2 * H, 4 * H)),
            _full((NLAYER_S2S - 1, H, 4 * H)),
            _full((NLAYER_S2S, H, 4 * H)),
            _full((NLAYER_S2S, 1, 4 * H)),
            _full((NLAYER_S2S, 1, 4 * H)),
            _full((2 * H, H)),
            _full((1, H)),
            _full((H, 1)),
            _full((1, 1)),
        ],
        out_specs=_full((1, 1)),
        out_shape=jax.ShapeDtypeStruct((1, 1), f32),
    )(
        h_nodes,
        rep8,
        bs,
        fold,
        lstm_Wih0.T,
        jnp.transpose(lstm_Wih12, (0, 2, 1)),
        jnp.transpose(lstm_Whh, (0, 2, 1)),
        lstm_bih.reshape(NLAYER_S2S, 1, 4 * H),
        lstm_bhh.reshape(NLAYER_S2S, 1, 4 * H),
        Wp1,
        bp1.reshape(1, H),
        Wp2,
        bp2.reshape(1, 1),
    )
    return out
